# baseline (speedup 1.0000x reference)
import functools

import jax
import jax.numpy as jnp
from jax.experimental import pallas as pl

_TILE_N = 512


def _kmeans_tile(x_ref, ct_ref, dist_ref, assign_ref, inertia_ref):
    x = x_ref[...]
    ct = ct_ref[...]
    k = ct.shape[1]
    cross2 = jax.lax.dot_general(
        x * -2.0, ct, (((1,), (0,)), ((), ())),
        preferred_element_type=jnp.float32)
    x_sq = jnp.sum(x * x, axis=1, keepdims=True)
    c_sq = jnp.sum(ct * ct, axis=0, keepdims=True)
    b = x_sq + c_sq
    d2 = jnp.maximum(b + cross2, 0.0)
    dist = jnp.sqrt(d2)
    dist_ref[...] = dist
    md = jnp.sqrt(jnp.min(d2, axis=1, keepdims=True))
    iota = jax.lax.broadcasted_iota(jnp.int32, (x.shape[0], k), 1)
    am = jnp.min(jnp.where(dist == md, iota, k), axis=1)
    assign_ref[...] = am[:, None]
    inertia_ref[...] = md * md


@functools.partial(jax.jit, static_argnames=())
def kernel(data, centroids):
    n, f = data.shape
    k = centroids.shape[0]
    grid = (n // _TILE_N,)
    dist, assign, inertia = pl.pallas_call(
        _kmeans_tile,
        grid=grid,
        in_specs=[
            pl.BlockSpec((_TILE_N, f), lambda i: (i, 0)),
            pl.BlockSpec((f, k), lambda i: (0, 0)),
        ],
        out_specs=[
            pl.BlockSpec((_TILE_N, k), lambda i: (i, 0)),
            pl.BlockSpec((_TILE_N, 1), lambda i: (i, 0)),
            pl.BlockSpec((_TILE_N, 1), lambda i: (i, 0)),
        ],
        out_shape=[
            jax.ShapeDtypeStruct((n, k), jnp.float32),
            jax.ShapeDtypeStruct((n, 1), jnp.int32),
            jax.ShapeDtypeStruct((n, 1), jnp.float32),
        ],
    )(data, centroids.T)
    return dist, assign[:, 0], inertia[:, 0]


# speedup vs baseline: 2.3384x; 2.3384x over previous
import functools

import jax
import jax.numpy as jnp
from jax.experimental import pallas as pl

_TILE_N = 512


def _kmeans_tile(x_ref, c_ref, distT_ref, assign_ref, inertia_ref):
    x = x_ref[...]
    c = c_ref[...]
    tn = x.shape[0]
    k = c.shape[0]
    xT = x.T
    cross2 = jax.lax.dot_general(
        c * -2.0, xT, (((1,), (0,)), ((), ())),
        preferred_element_type=jnp.float32)
    c_sq = jnp.sum(c * c, axis=1, keepdims=True)
    x_sq = jnp.sum(xT * xT, axis=0, keepdims=True)
    d2 = jnp.maximum((c_sq + x_sq) + cross2, 0.0)
    dist = jnp.sqrt(d2)
    distT_ref[...] = dist
    mn = jnp.min(d2, axis=0, keepdims=True)
    md = jnp.sqrt(mn)
    iota = jax.lax.broadcasted_iota(jnp.int32, (k, tn), 0)
    am = jnp.min(jnp.where(dist == md, iota, k), axis=0)
    assign_ref[...] = am[None, None, :]
    inertia_ref[...] = (md * md)[None]


@functools.partial(jax.jit, static_argnames=())
def kernel(data, centroids):
    n, f = data.shape
    k = centroids.shape[0]
    g = n // _TILE_N
    distT, assign, inertia = pl.pallas_call(
        _kmeans_tile,
        grid=(g,),
        in_specs=[
            pl.BlockSpec((_TILE_N, f), lambda i: (i, 0)),
            pl.BlockSpec((k, f), lambda i: (0, 0)),
        ],
        out_specs=[
            pl.BlockSpec((k, _TILE_N), lambda i: (0, i)),
            pl.BlockSpec((1, 1, _TILE_N), lambda i: (i, 0, 0)),
            pl.BlockSpec((1, 1, _TILE_N), lambda i: (i, 0, 0)),
        ],
        out_shape=[
            jax.ShapeDtypeStruct((k, n), jnp.float32),
            jax.ShapeDtypeStruct((g, 1, _TILE_N), jnp.int32),
            jax.ShapeDtypeStruct((g, 1, _TILE_N), jnp.float32),
        ],
    )(data, centroids)
    return distT.T, assign.reshape(n), inertia.reshape(n)


# speedup vs baseline: 2.5184x; 1.0770x over previous
import functools

import jax
import jax.numpy as jnp
from jax.experimental import pallas as pl
from jax.experimental.pallas import tpu as pltpu

_TILE_N = 512


def _kmeans_tile(x_ref, c_ref, distT_ref, assign_ref, inertia_ref,
                 c2_ref, csq_ref, iota_ref):
    tn = x_ref.shape[0]
    k = c_ref.shape[0]

    @pl.when(pl.program_id(0) == 0)
    def _prep():
        c = c_ref[...]
        c2_ref[...] = c * -2.0
        csq_ref[...] = jnp.sum(c * c, axis=1, keepdims=True)
        iota_ref[...] = jax.lax.broadcasted_iota(
            jnp.int32, (k, tn), 0).astype(jnp.float32)

    x = x_ref[...]
    xT = x.T
    cross2 = jax.lax.dot_general(
        c2_ref[...], xT, (((1,), (0,)), ((), ())),
        preferred_element_type=jnp.float32)
    x_sq = jnp.sum(xT * xT, axis=0, keepdims=True)
    d2 = jnp.maximum((csq_ref[...] + x_sq) + cross2, 0.0)
    dist = jnp.sqrt(d2)
    distT_ref[...] = dist
    mn = jnp.min(d2, axis=0, keepdims=True)
    md = jnp.sqrt(mn)
    am = jnp.min(jnp.where(dist == md, iota_ref[...], float(k)), axis=0)
    assign_ref[...] = am.astype(jnp.int32)[None, None, :]
    inertia_ref[...] = (md * md)[None]


@functools.partial(jax.jit, static_argnames=())
def kernel(data, centroids):
    n, f = data.shape
    k = centroids.shape[0]
    g = n // _TILE_N
    distT, assign, inertia = pl.pallas_call(
        _kmeans_tile,
        grid=(g,),
        in_specs=[
            pl.BlockSpec((_TILE_N, f), lambda i: (i, 0)),
            pl.BlockSpec((k, f), lambda i: (0, 0)),
        ],
        out_specs=[
            pl.BlockSpec((k, _TILE_N), lambda i: (0, i)),
            pl.BlockSpec((1, 1, _TILE_N), lambda i: (i, 0, 0)),
            pl.BlockSpec((1, 1, _TILE_N), lambda i: (i, 0, 0)),
        ],
        out_shape=[
            jax.ShapeDtypeStruct((k, n), jnp.float32),
            jax.ShapeDtypeStruct((g, 1, _TILE_N), jnp.int32),
            jax.ShapeDtypeStruct((g, 1, _TILE_N), jnp.float32),
        ],
        scratch_shapes=[
            pltpu.VMEM((k, f), jnp.float32),
            pltpu.VMEM((k, 1), jnp.float32),
            pltpu.VMEM((k, _TILE_N), jnp.float32),
        ],
    )(data, centroids)
    return distT.T, assign.reshape(n), inertia.reshape(n)


# speedup vs baseline: 2.9850x; 1.1853x over previous
import functools

import jax
import jax.numpy as jnp
from jax.experimental import pallas as pl
from jax.experimental.pallas import tpu as pltpu

_TILE_N = 1024


def _kmeans_tile(x_ref, c_ref, distT_ref, assign_ref, inertia_ref,
                 c2_ref, csq_ref, iota_ref):
    tn = x_ref.shape[0]
    k = c_ref.shape[0]

    @pl.when(pl.program_id(0) == 0)
    def _prep():
        c = c_ref[...]
        c2_ref[...] = c * -2.0
        csq_ref[...] = jnp.sum(c * c, axis=1, keepdims=True)
        iota_ref[...] = jax.lax.broadcasted_iota(
            jnp.int32, (k, tn), 0).astype(jnp.float32)

    x = x_ref[...]
    xT = x.T
    cross2 = jax.lax.dot_general(
        c2_ref[...], xT, (((1,), (0,)), ((), ())),
        preferred_element_type=jnp.float32)
    x_sq = jnp.sum(xT * xT, axis=0, keepdims=True)
    d2 = jnp.maximum((csq_ref[...] + x_sq) + cross2, 0.0)
    dist = jnp.sqrt(d2)
    distT_ref[...] = dist
    mn = jnp.min(d2, axis=0, keepdims=True)
    md = jnp.sqrt(mn)
    am = jnp.min(jnp.where(dist == md, iota_ref[...], float(k)), axis=0)
    assign_ref[...] = am.astype(jnp.int32)[None, None, :]
    inertia_ref[...] = (md * md)[None]


@functools.partial(jax.jit, static_argnames=())
def kernel(data, centroids):
    n, f = data.shape
    k = centroids.shape[0]
    g = n // _TILE_N
    distT, assign, inertia = pl.pallas_call(
        _kmeans_tile,
        grid=(g,),
        in_specs=[
            pl.BlockSpec((_TILE_N, f), lambda i: (i, 0)),
            pl.BlockSpec((k, f), lambda i: (0, 0)),
        ],
        out_specs=[
            pl.BlockSpec((k, _TILE_N), lambda i: (0, i)),
            pl.BlockSpec((1, 1, _TILE_N), lambda i: (i, 0, 0)),
            pl.BlockSpec((1, 1, _TILE_N), lambda i: (i, 0, 0)),
        ],
        out_shape=[
            jax.ShapeDtypeStruct((k, n), jnp.float32),
            jax.ShapeDtypeStruct((g, 1, _TILE_N), jnp.int32),
            jax.ShapeDtypeStruct((g, 1, _TILE_N), jnp.float32),
        ],
        scratch_shapes=[
            pltpu.VMEM((k, f), jnp.float32),
            pltpu.VMEM((k, 1), jnp.float32),
            pltpu.VMEM((k, _TILE_N), jnp.float32),
        ],
    )(data, centroids)
    return distT.T, assign.reshape(n), inertia.reshape(n)


# speedup vs baseline: 3.0283x; 1.0145x over previous
import functools

import jax
import jax.numpy as jnp
from jax.experimental import pallas as pl
from jax.experimental.pallas import tpu as pltpu

_TILE_N = 2048


def _kmeans_tile(x_ref, c_ref, distT_ref, assign_ref, inertia_ref,
                 c2_ref, csq_ref, iota_ref):
    tn = x_ref.shape[0]
    k = c_ref.shape[0]

    @pl.when(pl.program_id(0) == 0)
    def _prep():
        c = c_ref[...]
        c2_ref[...] = c * -2.0
        csq_ref[...] = jnp.sum(c * c, axis=1, keepdims=True)
        iota_ref[...] = jax.lax.broadcasted_iota(
            jnp.int32, (k, tn), 0).astype(jnp.float32)

    x = x_ref[...]
    xT = x.T
    cross2 = jax.lax.dot_general(
        c2_ref[...], xT, (((1,), (0,)), ((), ())),
        preferred_element_type=jnp.float32)
    x_sq = jnp.sum(xT * xT, axis=0, keepdims=True)
    d2 = jnp.maximum((csq_ref[...] + x_sq) + cross2, 0.0)
    dist = jnp.sqrt(d2)
    distT_ref[...] = dist
    mn = jnp.min(d2, axis=0, keepdims=True)
    md = jnp.sqrt(mn)
    am = jnp.min(jnp.where(dist == md, iota_ref[...], float(k)), axis=0)
    assign_ref[...] = am.astype(jnp.int32)[None, None, :]
    inertia_ref[...] = (md * md)[None]


@functools.partial(jax.jit, static_argnames=())
def kernel(data, centroids):
    n, f = data.shape
    k = centroids.shape[0]
    g = n // _TILE_N
    distT, assign, inertia = pl.pallas_call(
        _kmeans_tile,
        grid=(g,),
        in_specs=[
            pl.BlockSpec((_TILE_N, f), lambda i: (i, 0)),
            pl.BlockSpec((k, f), lambda i: (0, 0)),
        ],
        out_specs=[
            pl.BlockSpec((k, _TILE_N), lambda i: (0, i)),
            pl.BlockSpec((1, 1, _TILE_N), lambda i: (i, 0, 0)),
            pl.BlockSpec((1, 1, _TILE_N), lambda i: (i, 0, 0)),
        ],
        out_shape=[
            jax.ShapeDtypeStruct((k, n), jnp.float32),
            jax.ShapeDtypeStruct((g, 1, _TILE_N), jnp.int32),
            jax.ShapeDtypeStruct((g, 1, _TILE_N), jnp.float32),
        ],
        scratch_shapes=[
            pltpu.VMEM((k, f), jnp.float32),
            pltpu.VMEM((k, 1), jnp.float32),
            pltpu.VMEM((k, _TILE_N), jnp.float32),
        ],
    )(data, centroids)
    return distT.T, assign.reshape(n), inertia.reshape(n)
